# baseline (speedup 1.0000x reference)
import functools

import jax
import jax.numpy as jnp
from jax import lax
from jax.experimental import pallas as pl
from jax.experimental.pallas import tpu as pltpu
from jax.experimental.pallas import tpu_sc as plsc

N = 100000
E = 3200000
D = 10
DP = 16
NC = 2
NS = 16
NW = NC * NS
P = N // NW
CHUNK = 1600
BG = 128


def _tc_pre_body(x_ref, wy_ref, wz_ref, b1_ref, y_ref, z_ref):
    xb = x_ref[...]
    y_ref[...] = jnp.dot(xb, wy_ref[...], preferred_element_type=jnp.float32)
    z_ref[...] = (
        jnp.dot(xb, wz_ref[...], preferred_element_type=jnp.float32) + b1_ref[...]
    )


def _tc_pre(xp, wy, wz, b1p):
    br = 2000
    grid = N // br
    return pl.pallas_call(
        _tc_pre_body,
        grid=(grid,),
        in_specs=[
            pl.BlockSpec((br, DP), lambda i: (i, 0)),
            pl.BlockSpec((DP, DP), lambda i: (0, 0)),
            pl.BlockSpec((DP, DP), lambda i: (0, 0)),
            pl.BlockSpec((1, DP), lambda i: (0, 0)),
        ],
        out_specs=[
            pl.BlockSpec((br, DP), lambda i: (i, 0)),
            pl.BlockSpec((br, DP), lambda i: (i, 0)),
        ],
        out_shape=[
            jax.ShapeDtypeStruct((N, DP), jnp.float32),
            jax.ShapeDtypeStruct((N, DP), jnp.float32),
        ],
    )(xp, wy, wz, b1p)


def _tc_post_body(m_ref, z_ref, w2_ref, b2_ref, o_ref):
    m = m_ref[...]
    agg = jnp.where(m == -jnp.inf, 0.0, z_ref[...] + m)
    o_ref[...] = (
        jnp.dot(agg, w2_ref[...], preferred_element_type=jnp.float32) + b2_ref[...]
    )


def _tc_post(m2, z16, w2p, b2p):
    br = 2000
    grid = N // br
    return pl.pallas_call(
        _tc_post_body,
        grid=(grid,),
        in_specs=[
            pl.BlockSpec((br, DP), lambda i: (i, 0)),
            pl.BlockSpec((br, DP), lambda i: (i, 0)),
            pl.BlockSpec((DP, DP), lambda i: (0, 0)),
            pl.BlockSpec((1, DP), lambda i: (0, 0)),
        ],
        out_specs=pl.BlockSpec((br, DP), lambda i: (i, 0)),
        out_shape=jax.ShapeDtypeStruct((N, DP), jnp.float32),
    )(m2, z16, w2p, b2p)


NCHUNK = E // CHUNK
WAVE = 2
COMP = CHUNK + 2 * BG
UNROLL = 4


def _sc_body(
    y_hbm, dst_hbm, src_hbm, m_hbm,
    acc0, acc1, dstca, srcca, dstcb, srccb,
    dlca, slca, dlcb, slcb, rowsa, rowsb,
    semca, semcb, semga, semgb,
):
    cid = lax.axis_index("c")
    sid = lax.axis_index("s")
    wid = sid * NC + cid
    base = wid * P
    iota = lax.iota(jnp.int32, 16)
    neg_inf = jnp.full((16,), -jnp.inf, jnp.float32)
    dump_row = jnp.full((16,), P, jnp.int32)
    ones = jnp.full((16,), 1, jnp.int32)
    zeros = jnp.full((16,), 0, jnp.int32)
    pu = jnp.uint32(P)

    def init_body(i, _):
        plsc.store_scatter(acc0, [i * 16 + iota], neg_inf)
        plsc.store_scatter(acc1, [i * 16 + iota], neg_inf)
        return 0

    lax.fori_loop(0, P + 1, init_body, 0)

    def fire_chunk(c, dref, sref, semc):
        off = c * CHUNK
        pltpu.async_copy(dst_hbm.at[pl.ds(off, CHUNK)], dref, semc)
        pltpu.async_copy(src_hbm.at[pl.ds(off, CHUNK)], sref, semc)

    def wait_chunk(c, dref, sref, semc):
        off = c * CHUNK
        pltpu.make_async_copy(dst_hbm.at[pl.ds(off, CHUNK)], dref, semc).wait()
        pltpu.make_async_copy(src_hbm.at[pl.ds(off, CHUNK)], sref, semc).wait()

    def scan_chunk(dref, sref, dlc, slc):
        def scan_body(g4, cnt):
            g = g4 * (16 * UNROLL)
            parts = []
            for k in range(UNROLL):
                d16 = dref[pl.ds(g + k * 16, 16)]
                s16 = sref[pl.ds(g + k * 16, 16)]
                dl = d16 - base
                msk = dl.astype(jnp.uint32) < pu
                mi = jnp.where(msk, ones, zeros)
                pos = plsc.cumsum(mi)
                tot = jnp.sum(mi)
                parts.append((dl, s16, msk, pos, tot))
            for dl, s16, msk, pos, tot in parts:
                offs = pos + (cnt - 1)
                plsc.store_scatter(dlc, [offs], dl, mask=msk)
                plsc.store_scatter(slc, [offs], s16, mask=msk)
                cnt = cnt + tot
            return cnt

        return lax.fori_loop(0, CHUNK // (16 * UNROLL), scan_body, 0)

    def dump_fill(cnt, dlc, slc):
        def dump_body(j, _):
            idx16 = cnt + j * 16 + iota
            plsc.store_scatter(dlc, [idx16], dump_row)
            plsc.store_scatter(slc, [idx16], (idx16 * 1237 + wid * 61) & 0xFFFF)
            return 0

        lax.fori_loop(0, BG // 16, dump_body, 0)

    def fire_gathers(nblk, slc, rows, semg):
        nb = jnp.minimum(nblk, WAVE)

        def fire_blk(k, _):
            pltpu.async_copy(
                y_hbm.at[slc.at[pl.ds(k * BG, BG)]],
                rows.at[pl.ds(k * BG, BG)],
                semg,
            )
            return 0

        lax.fori_loop(0, nb, fire_blk, 0)

    def rmw_edges(b0, nb, dlc, rows):
        def edge_body(j, _):
            e = b0 * BG + 2 * j
            dl0 = plsc.load_gather(dlc, [jnp.full((16,), e, jnp.int32)])
            dl1 = plsc.load_gather(dlc, [jnp.full((16,), e + 1, jnp.int32)])
            msg0 = rows[2 * j]
            msg1 = rows[2 * j + 1]
            f0 = dl0 * 16 + iota
            f1 = dl1 * 16 + iota
            cur0 = plsc.load_gather(acc0, [f0])
            cur1 = plsc.load_gather(acc1, [f1])
            plsc.store_scatter(acc0, [f0], jnp.maximum(cur0, msg0))
            plsc.store_scatter(acc1, [f1], jnp.maximum(cur1, msg1))
            return 0

        lax.fori_loop(0, nb * (BG // 2), edge_body, 0)

    def drain_and_rmw(cnt, dlc, slc, rows, semg):
        nblk = (cnt + BG - 1) // BG
        nb0 = jnp.minimum(nblk, WAVE)

        def drain_blk(k, _):
            pltpu.make_async_copy(
                y_hbm.at[slc.at[pl.ds(k * BG, BG)]],
                rows.at[pl.ds(k * BG, BG)],
                semg,
            ).wait()
            return 0

        lax.fori_loop(0, nb0, drain_blk, 0)
        rmw_edges(0, nb0, dlc, rows)

        def over_body(b, _):
            pltpu.async_copy(
                y_hbm.at[slc.at[pl.ds(b * BG, BG)]],
                rows.at[pl.ds(0, BG)],
                semg,
            ).wait()
            rmw_edges_one(b, rows, dlc)
            return 0

        def rmw_edges_one(b, rows_, dlc_):
            def edge_body(j, _):
                e = b * BG + 2 * j
                dl0 = plsc.load_gather(dlc_, [jnp.full((16,), e, jnp.int32)])
                dl1 = plsc.load_gather(dlc_, [jnp.full((16,), e + 1, jnp.int32)])
                msg0 = rows_[2 * j]
                msg1 = rows_[2 * j + 1]
                f0 = dl0 * 16 + iota
                f1 = dl1 * 16 + iota
                cur0 = plsc.load_gather(acc0, [f0])
                cur1 = plsc.load_gather(acc1, [f1])
                plsc.store_scatter(acc0, [f0], jnp.maximum(cur0, msg0))
                plsc.store_scatter(acc1, [f1], jnp.maximum(cur1, msg1))
                return 0

            lax.fori_loop(0, BG // 2, edge_body, 0)

        lax.fori_loop(WAVE, nblk, over_body, 0)

    fire_chunk(0, dstca, srcca, semca)
    fire_chunk(1, dstcb, srccb, semcb)

    def pair_body(t, cnt_b):
        ca = 2 * t
        wait_chunk(ca, dstca, srcca, semca)
        cnt_a = scan_chunk(dstca, srcca, dlca, slca)
        dump_fill(cnt_a, dlca, slca)
        fire_gathers((cnt_a + BG - 1) // BG, slca, rowsa, semga)

        @pl.when(t < NCHUNK // 2 - 1)
        def _():
            fire_chunk(ca + 2, dstca, srcca, semca)

        drain_and_rmw(cnt_b, dlcb, slcb, rowsb, semgb)

        cb = 2 * t + 1
        wait_chunk(cb, dstcb, srccb, semcb)
        cnt_b = scan_chunk(dstcb, srccb, dlcb, slcb)
        dump_fill(cnt_b, dlcb, slcb)
        fire_gathers((cnt_b + BG - 1) // BG, slcb, rowsb, semgb)

        @pl.when(t < NCHUNK // 2 - 1)
        def _():
            fire_chunk(cb + 2, dstcb, srccb, semcb)

        drain_and_rmw(cnt_a, dlca, slca, rowsa, semga)
        return cnt_b

    cnt_last = lax.fori_loop(0, NCHUNK // 2, pair_body, 0)
    drain_and_rmw(cnt_last, dlcb, slcb, rowsb, semgb)

    def merge_body(i, _):
        a0 = acc0[pl.ds(i * 16, 16)]
        a1 = acc1[pl.ds(i * 16, 16)]
        acc0[pl.ds(i * 16, 16)] = jnp.maximum(a0, a1)
        return 0

    lax.fori_loop(0, P, merge_body, 0)

    pltpu.sync_copy(acc0.at[pl.ds(0, P * DP)], m_hbm.at[wid])


@functools.partial(
    pl.kernel,
    out_type=jax.ShapeDtypeStruct((NW, P * DP), jnp.float32),
    mesh=plsc.VectorSubcoreMesh(core_axis_name="c", subcore_axis_name="s"),
    compiler_params=pltpu.CompilerParams(
        needs_layout_passes=False, use_tc_tiling_on_sc=False
    ),
    scratch_types=[
        pltpu.VMEM(((P + 1) * DP,), jnp.float32),
        pltpu.VMEM(((P + 1) * DP,), jnp.float32),
        pltpu.VMEM((CHUNK,), jnp.int32),
        pltpu.VMEM((CHUNK,), jnp.int32),
        pltpu.VMEM((CHUNK,), jnp.int32),
        pltpu.VMEM((CHUNK,), jnp.int32),
        pltpu.VMEM((COMP,), jnp.int32),
        pltpu.VMEM((COMP,), jnp.int32),
        pltpu.VMEM((COMP,), jnp.int32),
        pltpu.VMEM((COMP,), jnp.int32),
        pltpu.VMEM((WAVE * BG, DP), jnp.float32),
        pltpu.VMEM((WAVE * BG, DP), jnp.float32),
        pltpu.SemaphoreType.DMA,
        pltpu.SemaphoreType.DMA,
        pltpu.SemaphoreType.DMA,
        pltpu.SemaphoreType.DMA,
    ],
)
def _sc_segmax(
    y_hbm, dst_hbm, src_hbm, m_hbm,
    acc0, acc1, dstca, srcca, dstcb, srccb,
    dlca, slca, dlcb, slcb, rowsa, rowsb,
    semca, semcb, semga, semgb,
):
    _sc_body(
        y_hbm, dst_hbm, src_hbm, m_hbm,
        acc0, acc1, dstca, srcca, dstcb, srccb,
        dlca, slca, dlcb, slcb, rowsa, rowsb,
        semca, semcb, semga, semgb,
    )


@jax.jit
def kernel(x, edge_index, mask, W1, b1, W2, b2):
    del mask
    w1a = W1[:, :D]
    w1b = W1[:, D:]
    wy = jnp.zeros((DP, DP), jnp.float32).at[:D, :D].set(w1b.T)
    wz = jnp.zeros((DP, DP), jnp.float32).at[:D, :D].set((w1a - w1b).T)
    b1p = jnp.zeros((1, DP), jnp.float32).at[0, :D].set(b1)
    w2p = jnp.zeros((DP, DP), jnp.float32).at[:D, :D].set(W2.T)
    b2p = jnp.zeros((1, DP), jnp.float32).at[0, :D].set(b2)
    xp = jnp.pad(x, ((0, 0), (0, DP - D)))

    y16, z16 = _tc_pre(xp, wy, wz, b1p)
    src = edge_index[0].astype(jnp.int32)
    dst = edge_index[1].astype(jnp.int32)
    m = _sc_segmax(y16, dst, src)
    out16 = _tc_post(m.reshape(N, DP), z16, w2p, b2p)
    return out16[:, :D]


# speedup vs baseline: 1.4321x; 1.4321x over previous
import functools

import jax
import jax.numpy as jnp
from jax import lax
from jax.experimental import pallas as pl
from jax.experimental.pallas import tpu as pltpu
from jax.experimental.pallas import tpu_sc as plsc

N = 100000
E = 3200000
D = 10
DP = 16
NC = 2
NS = 16
NW = NC * NS
P = N // NW
CHUNK = 3200
BG = 128


def _tc_pre_body(x_ref, wy_ref, wz_ref, b1_ref, y_ref, z_ref):
    xb = x_ref[...]
    y_ref[...] = jnp.dot(xb, wy_ref[...], preferred_element_type=jnp.float32)
    z_ref[...] = (
        jnp.dot(xb, wz_ref[...], preferred_element_type=jnp.float32) + b1_ref[...]
    )


def _tc_pre(xp, wy, wz, b1p):
    br = 2000
    grid = N // br
    return pl.pallas_call(
        _tc_pre_body,
        grid=(grid,),
        in_specs=[
            pl.BlockSpec((br, DP), lambda i: (i, 0)),
            pl.BlockSpec((DP, DP), lambda i: (0, 0)),
            pl.BlockSpec((DP, DP), lambda i: (0, 0)),
            pl.BlockSpec((1, DP), lambda i: (0, 0)),
        ],
        out_specs=[
            pl.BlockSpec((br, DP), lambda i: (i, 0)),
            pl.BlockSpec((br, DP), lambda i: (i, 0)),
        ],
        out_shape=[
            jax.ShapeDtypeStruct((N, DP), jnp.float32),
            jax.ShapeDtypeStruct((N, DP), jnp.float32),
        ],
    )(xp, wy, wz, b1p)


def _tc_post_body(m_ref, z_ref, w2_ref, b2_ref, o_ref):
    m = m_ref[...]
    agg = jnp.where(m == -jnp.inf, 0.0, z_ref[...] + m)
    o_ref[...] = (
        jnp.dot(agg, w2_ref[...], preferred_element_type=jnp.float32) + b2_ref[...]
    )


def _tc_post(m2, z16, w2p, b2p):
    br = 2000
    grid = N // br
    return pl.pallas_call(
        _tc_post_body,
        grid=(grid,),
        in_specs=[
            pl.BlockSpec((br, DP), lambda i: (i, 0)),
            pl.BlockSpec((br, DP), lambda i: (i, 0)),
            pl.BlockSpec((DP, DP), lambda i: (0, 0)),
            pl.BlockSpec((1, DP), lambda i: (0, 0)),
        ],
        out_specs=pl.BlockSpec((br, DP), lambda i: (i, 0)),
        out_shape=jax.ShapeDtypeStruct((N, DP), jnp.float32),
    )(m2, z16, w2p, b2p)


NCHUNK = E // CHUNK
WAVE = 1
COMP = CHUNK + 2 * BG
UNROLL = 4


def _sc_body(
    y_hbm, dst_hbm, src_hbm, m_hbm,
    acc0, acc1, dstca, srcca, dstcb, srccb,
    dlca, slca, dlcb, slcb, rowsa, rowsb,
    semca, semcb, semga, semgb,
):
    cid = lax.axis_index("c")
    sid = lax.axis_index("s")
    wid = sid * NC + cid
    base = wid * P
    iota = lax.iota(jnp.int32, 16)
    neg_inf = jnp.full((16,), -jnp.inf, jnp.float32)
    dump_row = jnp.full((16,), P, jnp.int32)
    ones = jnp.full((16,), 1, jnp.int32)
    zeros = jnp.full((16,), 0, jnp.int32)
    pu = jnp.uint32(P)

    def init_body(i, _):
        plsc.store_scatter(acc0, [i * 16 + iota], neg_inf)
        plsc.store_scatter(acc1, [i * 16 + iota], neg_inf)
        return 0

    lax.fori_loop(0, P + 1, init_body, 0)

    def fire_chunk(c, dref, sref, semc):
        off = c * CHUNK
        pltpu.async_copy(dst_hbm.at[pl.ds(off, CHUNK)], dref, semc)
        pltpu.async_copy(src_hbm.at[pl.ds(off, CHUNK)], sref, semc)

    def wait_chunk(c, dref, sref, semc):
        off = c * CHUNK
        pltpu.make_async_copy(dst_hbm.at[pl.ds(off, CHUNK)], dref, semc).wait()
        pltpu.make_async_copy(src_hbm.at[pl.ds(off, CHUNK)], sref, semc).wait()

    def scan_chunk(dref, sref, dlc, slc):
        def scan_body(g4, cnt):
            g = g4 * (16 * UNROLL)
            parts = []
            for k in range(UNROLL):
                d16 = dref[pl.ds(g + k * 16, 16)]
                s16 = sref[pl.ds(g + k * 16, 16)]
                dl = d16 - base
                msk = dl.astype(jnp.uint32) < pu
                mi = jnp.where(msk, ones, zeros)
                pos = plsc.cumsum(mi)
                tot = jnp.sum(mi)
                parts.append((dl, s16, msk, pos, tot))
            for dl, s16, msk, pos, tot in parts:
                offs = pos + (cnt - 1)
                plsc.store_scatter(dlc, [offs], dl, mask=msk)
                plsc.store_scatter(slc, [offs], s16, mask=msk)
                cnt = cnt + tot
            return cnt

        return lax.fori_loop(0, CHUNK // (16 * UNROLL), scan_body, 0)

    def dump_fill(cnt, dlc, slc):
        def dump_body(j, _):
            idx16 = cnt + j * 16 + iota
            plsc.store_scatter(dlc, [idx16], dump_row)
            plsc.store_scatter(slc, [idx16], (idx16 * 1237 + wid * 61) & 0xFFFF)
            return 0

        lax.fori_loop(0, BG // 16, dump_body, 0)

    def fire_gathers(nblk, slc, rows, semg):
        nb = jnp.minimum(nblk, WAVE)

        def fire_blk(k, _):
            pltpu.async_copy(
                y_hbm.at[slc.at[pl.ds(k * BG, BG)]],
                rows.at[pl.ds(k * BG, BG)],
                semg,
            )
            return 0

        lax.fori_loop(0, nb, fire_blk, 0)

    def rmw_edges(b0, nb, cnt, dlc, rows):
        npairs = (jnp.clip(cnt - b0 * BG, 0, nb * BG) + 1) // 2

        def edge_body(j, _):
            e = b0 * BG + 2 * j
            dl0 = plsc.load_gather(dlc, [jnp.full((16,), e, jnp.int32)])
            dl1 = plsc.load_gather(dlc, [jnp.full((16,), e + 1, jnp.int32)])
            msg0 = rows[2 * j]
            msg1 = rows[2 * j + 1]
            f0 = dl0 * 16 + iota
            f1 = dl1 * 16 + iota
            cur0 = plsc.load_gather(acc0, [f0])
            cur1 = plsc.load_gather(acc1, [f1])
            plsc.store_scatter(acc0, [f0], jnp.maximum(cur0, msg0))
            plsc.store_scatter(acc1, [f1], jnp.maximum(cur1, msg1))
            return 0

        lax.fori_loop(0, npairs, edge_body, 0)

    def drain_and_rmw(cnt, dlc, slc, rows, semg):
        nblk = (cnt + BG - 1) // BG
        nb0 = jnp.minimum(nblk, WAVE)

        def drain_blk(k, _):
            pltpu.make_async_copy(
                y_hbm.at[slc.at[pl.ds(k * BG, BG)]],
                rows.at[pl.ds(k * BG, BG)],
                semg,
            ).wait()
            return 0

        lax.fori_loop(0, nb0, drain_blk, 0)
        rmw_edges(0, nb0, cnt, dlc, rows)

        def over_body(b, _):
            pltpu.async_copy(
                y_hbm.at[slc.at[pl.ds(b * BG, BG)]],
                rows.at[pl.ds(0, BG)],
                semg,
            ).wait()
            npairs = (jnp.clip(cnt - b * BG, 0, BG) + 1) // 2

            def edge_body(j, _):
                e = b * BG + 2 * j
                dl0 = plsc.load_gather(dlc, [jnp.full((16,), e, jnp.int32)])
                dl1 = plsc.load_gather(dlc, [jnp.full((16,), e + 1, jnp.int32)])
                msg0 = rows[2 * j]
                msg1 = rows[2 * j + 1]
                f0 = dl0 * 16 + iota
                f1 = dl1 * 16 + iota
                cur0 = plsc.load_gather(acc0, [f0])
                cur1 = plsc.load_gather(acc1, [f1])
                plsc.store_scatter(acc0, [f0], jnp.maximum(cur0, msg0))
                plsc.store_scatter(acc1, [f1], jnp.maximum(cur1, msg1))
                return 0

            lax.fori_loop(0, npairs, edge_body, 0)
            return 0

        lax.fori_loop(WAVE, nblk, over_body, 0)

    fire_chunk(0, dstca, srcca, semca)
    fire_chunk(1, dstcb, srccb, semcb)

    def pair_body(t, cnt_b):
        ca = 2 * t
        wait_chunk(ca, dstca, srcca, semca)
        cnt_a = scan_chunk(dstca, srcca, dlca, slca)
        dump_fill(cnt_a, dlca, slca)
        fire_gathers((cnt_a + BG - 1) // BG, slca, rowsa, semga)

        @pl.when(t < NCHUNK // 2 - 1)
        def _():
            fire_chunk(ca + 2, dstca, srcca, semca)

        drain_and_rmw(cnt_b, dlcb, slcb, rowsb, semgb)

        cb = 2 * t + 1
        wait_chunk(cb, dstcb, srccb, semcb)
        cnt_b = scan_chunk(dstcb, srccb, dlcb, slcb)
        dump_fill(cnt_b, dlcb, slcb)
        fire_gathers((cnt_b + BG - 1) // BG, slcb, rowsb, semgb)

        @pl.when(t < NCHUNK // 2 - 1)
        def _():
            fire_chunk(cb + 2, dstcb, srccb, semcb)

        drain_and_rmw(cnt_a, dlca, slca, rowsa, semga)
        return cnt_b

    cnt_last = lax.fori_loop(0, NCHUNK // 2, pair_body, 0)
    drain_and_rmw(cnt_last, dlcb, slcb, rowsb, semgb)

    def merge_body(i, _):
        a0 = acc0[pl.ds(i * 16, 16)]
        a1 = acc1[pl.ds(i * 16, 16)]
        acc0[pl.ds(i * 16, 16)] = jnp.maximum(a0, a1)
        return 0

    lax.fori_loop(0, P, merge_body, 0)

    pltpu.sync_copy(acc0.at[pl.ds(0, P * DP)], m_hbm.at[wid])


@functools.partial(
    pl.kernel,
    out_type=jax.ShapeDtypeStruct((NW, P * DP), jnp.float32),
    mesh=plsc.VectorSubcoreMesh(core_axis_name="c", subcore_axis_name="s"),
    compiler_params=pltpu.CompilerParams(
        needs_layout_passes=False, use_tc_tiling_on_sc=False
    ),
    scratch_types=[
        pltpu.VMEM(((P + 1) * DP,), jnp.float32),
        pltpu.VMEM(((P + 1) * DP,), jnp.float32),
        pltpu.VMEM((CHUNK,), jnp.int32),
        pltpu.VMEM((CHUNK,), jnp.int32),
        pltpu.VMEM((CHUNK,), jnp.int32),
        pltpu.VMEM((CHUNK,), jnp.int32),
        pltpu.VMEM((COMP,), jnp.int32),
        pltpu.VMEM((COMP,), jnp.int32),
        pltpu.VMEM((COMP,), jnp.int32),
        pltpu.VMEM((COMP,), jnp.int32),
        pltpu.VMEM((WAVE * BG, DP), jnp.float32),
        pltpu.VMEM((WAVE * BG, DP), jnp.float32),
        pltpu.SemaphoreType.DMA,
        pltpu.SemaphoreType.DMA,
        pltpu.SemaphoreType.DMA,
        pltpu.SemaphoreType.DMA,
    ],
)
def _sc_segmax(
    y_hbm, dst_hbm, src_hbm, m_hbm,
    acc0, acc1, dstca, srcca, dstcb, srccb,
    dlca, slca, dlcb, slcb, rowsa, rowsb,
    semca, semcb, semga, semgb,
):
    _sc_body(
        y_hbm, dst_hbm, src_hbm, m_hbm,
        acc0, acc1, dstca, srcca, dstcb, srccb,
        dlca, slca, dlcb, slcb, rowsa, rowsb,
        semca, semcb, semga, semgb,
    )


@jax.jit
def kernel(x, edge_index, mask, W1, b1, W2, b2):
    del mask
    w1a = W1[:, :D]
    w1b = W1[:, D:]
    wy = jnp.zeros((DP, DP), jnp.float32).at[:D, :D].set(w1b.T)
    wz = jnp.zeros((DP, DP), jnp.float32).at[:D, :D].set((w1a - w1b).T)
    b1p = jnp.zeros((1, DP), jnp.float32).at[0, :D].set(b1)
    w2p = jnp.zeros((DP, DP), jnp.float32).at[:D, :D].set(W2.T)
    b2p = jnp.zeros((1, DP), jnp.float32).at[0, :D].set(b2)
    xp = jnp.pad(x, ((0, 0), (0, DP - D)))

    y16, z16 = _tc_pre(xp, wy, wz, b1p)
    src = edge_index[0].astype(jnp.int32)
    dst = edge_index[1].astype(jnp.int32)
    m = _sc_segmax(y16, dst, src)
    out16 = _tc_post(m.reshape(N, DP), z16, w2p, b2p)
    return out16[:, :D]


# speedup vs baseline: 2.0368x; 1.4223x over previous
import functools

import jax
import jax.numpy as jnp
from jax import lax
from jax.experimental import pallas as pl
from jax.experimental.pallas import tpu as pltpu
from jax.experimental.pallas import tpu_sc as plsc

N = 100000
E = 3200000
D = 10
DP = 16
NC = 2
NS = 16
NW = NC * NS
P = N // NW
CHUNK = 3200
BG = 128


def _tc_pre_body(x_ref, wy_ref, wz_ref, b1_ref, y_ref, z_ref):
    xb = x_ref[...]
    y_ref[...] = jnp.dot(xb, wy_ref[...], preferred_element_type=jnp.float32)
    z_ref[...] = (
        jnp.dot(xb, wz_ref[...], preferred_element_type=jnp.float32) + b1_ref[...]
    )


def _tc_pre(xp, wy, wz, b1p):
    br = 2000
    grid = N // br
    return pl.pallas_call(
        _tc_pre_body,
        grid=(grid,),
        in_specs=[
            pl.BlockSpec((br, DP), lambda i: (i, 0)),
            pl.BlockSpec((DP, DP), lambda i: (0, 0)),
            pl.BlockSpec((DP, DP), lambda i: (0, 0)),
            pl.BlockSpec((1, DP), lambda i: (0, 0)),
        ],
        out_specs=[
            pl.BlockSpec((br, DP), lambda i: (i, 0)),
            pl.BlockSpec((br, DP), lambda i: (i, 0)),
        ],
        out_shape=[
            jax.ShapeDtypeStruct((N, DP), jnp.float32),
            jax.ShapeDtypeStruct((N, DP), jnp.float32),
        ],
    )(xp, wy, wz, b1p)


def _tc_post_body(m_ref, z_ref, w2_ref, b2_ref, o_ref):
    m = m_ref[...]
    agg = jnp.where(m == -jnp.inf, 0.0, z_ref[...] + m)
    o_ref[...] = (
        jnp.dot(agg, w2_ref[...], preferred_element_type=jnp.float32) + b2_ref[...]
    )


def _tc_post(m2, z16, w2p, b2p):
    br = 2000
    grid = N // br
    return pl.pallas_call(
        _tc_post_body,
        grid=(grid,),
        in_specs=[
            pl.BlockSpec((br, DP), lambda i: (i, 0)),
            pl.BlockSpec((br, DP), lambda i: (i, 0)),
            pl.BlockSpec((DP, DP), lambda i: (0, 0)),
            pl.BlockSpec((1, DP), lambda i: (0, 0)),
        ],
        out_specs=pl.BlockSpec((br, DP), lambda i: (i, 0)),
        out_shape=jax.ShapeDtypeStruct((N, DP), jnp.float32),
    )(m2, z16, w2p, b2p)


NCHUNK = E // CHUNK
WAVE = 1
COMP = CHUNK + 2 * BG
UNROLL = 8


def _sc_body(
    y_hbm, dst_hbm, src_hbm, m_hbm,
    acc0, acc1, dstca, srcca, dstcb, srccb,
    dlca, slca, dlcb, slcb, rowsa, rowsb,
    semca, semcb, semga, semgb,
):
    cid = lax.axis_index("c")
    sid = lax.axis_index("s")
    wid = sid * NC + cid
    base = wid * P
    iota = lax.iota(jnp.int32, 16)
    neg_inf = jnp.full((16,), -jnp.inf, jnp.float32)
    dump_row = jnp.full((16,), P, jnp.int32)
    ones = jnp.full((16,), 1, jnp.int32)
    zeros = jnp.full((16,), 0, jnp.int32)
    pu = jnp.uint32(P)

    def init_body(i, _):
        plsc.store_scatter(acc0, [i * 16 + iota], neg_inf)
        plsc.store_scatter(acc1, [i * 16 + iota], neg_inf)
        return 0

    lax.fori_loop(0, P + 1, init_body, 0)

    def fire_chunk(c, dref, sref, semc):
        off = c * CHUNK
        pltpu.async_copy(dst_hbm.at[pl.ds(off, CHUNK)], dref, semc)
        pltpu.async_copy(src_hbm.at[pl.ds(off, CHUNK)], sref, semc)

    def wait_chunk(c, dref, sref, semc):
        off = c * CHUNK
        pltpu.make_async_copy(dst_hbm.at[pl.ds(off, CHUNK)], dref, semc).wait()
        pltpu.make_async_copy(src_hbm.at[pl.ds(off, CHUNK)], sref, semc).wait()

    lane15 = jnp.full((16,), 15, jnp.int32)

    def bcast_last(vec):
        return lax.gather(
            vec,
            lane15[:, None],
            dimension_numbers=lax.GatherDimensionNumbers(
                offset_dims=(), collapsed_slice_dims=(0,), start_index_map=(0,)
            ),
            slice_sizes=(1,),
            mode=lax.GatherScatterMode.PROMISE_IN_BOUNDS,
        )

    def scan_chunk(dref, sref, dlc, slc):
        def scan_body(g4, cntv):
            g = g4 * (16 * UNROLL)
            parts = []
            for k in range(UNROLL):
                d16 = dref[pl.ds(g + k * 16, 16)]
                s16 = sref[pl.ds(g + k * 16, 16)]
                dl = d16 - base
                msk = dl.astype(jnp.uint32) < pu
                mi = jnp.where(msk, ones, zeros)
                pos = plsc.cumsum(mi)
                parts.append((dl, s16, msk, pos))
            for dl, s16, msk, pos in parts:
                offs = pos + cntv
                plsc.store_scatter(dlc, [offs], dl, mask=msk)
                plsc.store_scatter(slc, [offs], s16, mask=msk)
                cntv = cntv + bcast_last(pos)
            return cntv

        cntv = lax.fori_loop(
            0, CHUNK // (16 * UNROLL), scan_body,
            jnp.full((16,), -1, jnp.int32),
        )
        return jnp.max(cntv) + 1

    def dump_fill(cnt, dlc, slc):
        def dump_body(j, _):
            idx16 = cnt + j * 16 + iota
            plsc.store_scatter(dlc, [idx16], dump_row)
            plsc.store_scatter(slc, [idx16], (idx16 * 1237 + wid * 61) & 0xFFFF)
            return 0

        lax.fori_loop(0, BG // 16, dump_body, 0)

    def fire_gathers(nblk, slc, rows, semg):
        nb = jnp.minimum(nblk, WAVE)

        def fire_blk(k, _):
            pltpu.async_copy(
                y_hbm.at[slc.at[pl.ds(k * BG, BG)]],
                rows.at[pl.ds(k * BG, BG)],
                semg,
            )
            return 0

        lax.fori_loop(0, nb, fire_blk, 0)

    def rmw_edges(b0, nb, cnt, dlc, rows):
        npairs = (jnp.clip(cnt - b0 * BG, 0, nb * BG) + 1) // 2
        nquad = (npairs + 1) // 2

        def edge_body(j, _):
            for q in range(2):
                e = b0 * BG + 4 * j + 2 * q
                r = 4 * j + 2 * q
                dl0 = plsc.load_gather(dlc, [jnp.full((16,), e, jnp.int32)])
                dl1 = plsc.load_gather(dlc, [jnp.full((16,), e + 1, jnp.int32)])
                msg0 = rows[r]
                msg1 = rows[r + 1]
                f0 = dl0 * 16 + iota
                f1 = dl1 * 16 + iota
                cur0 = plsc.load_gather(acc0, [f0])
                cur1 = plsc.load_gather(acc1, [f1])
                plsc.store_scatter(acc0, [f0], jnp.maximum(cur0, msg0))
                plsc.store_scatter(acc1, [f1], jnp.maximum(cur1, msg1))
            return 0

        lax.fori_loop(0, nquad, edge_body, 0)

    def drain_and_rmw(cnt, dlc, slc, rows, semg):
        nblk = (cnt + BG - 1) // BG
        nb0 = jnp.minimum(nblk, WAVE)

        def drain_blk(k, _):
            pltpu.make_async_copy(
                y_hbm.at[slc.at[pl.ds(k * BG, BG)]],
                rows.at[pl.ds(k * BG, BG)],
                semg,
            ).wait()
            return 0

        lax.fori_loop(0, nb0, drain_blk, 0)
        rmw_edges(0, nb0, cnt, dlc, rows)

        def over_body(b, _):
            pltpu.async_copy(
                y_hbm.at[slc.at[pl.ds(b * BG, BG)]],
                rows.at[pl.ds(0, BG)],
                semg,
            ).wait()
            npairs = (jnp.clip(cnt - b * BG, 0, BG) + 1) // 2

            def edge_body(j, _):
                e = b * BG + 2 * j
                dl0 = plsc.load_gather(dlc, [jnp.full((16,), e, jnp.int32)])
                dl1 = plsc.load_gather(dlc, [jnp.full((16,), e + 1, jnp.int32)])
                msg0 = rows[2 * j]
                msg1 = rows[2 * j + 1]
                f0 = dl0 * 16 + iota
                f1 = dl1 * 16 + iota
                cur0 = plsc.load_gather(acc0, [f0])
                cur1 = plsc.load_gather(acc1, [f1])
                plsc.store_scatter(acc0, [f0], jnp.maximum(cur0, msg0))
                plsc.store_scatter(acc1, [f1], jnp.maximum(cur1, msg1))
                return 0

            lax.fori_loop(0, npairs, edge_body, 0)
            return 0

        lax.fori_loop(WAVE, nblk, over_body, 0)

    fire_chunk(0, dstca, srcca, semca)
    fire_chunk(1, dstcb, srccb, semcb)

    def pair_body(t, cnt_b):
        ca = 2 * t
        wait_chunk(ca, dstca, srcca, semca)
        cnt_a = scan_chunk(dstca, srcca, dlca, slca)
        dump_fill(cnt_a, dlca, slca)
        fire_gathers((cnt_a + BG - 1) // BG, slca, rowsa, semga)

        @pl.when(t < NCHUNK // 2 - 1)
        def _():
            fire_chunk(ca + 2, dstca, srcca, semca)

        drain_and_rmw(cnt_b, dlcb, slcb, rowsb, semgb)

        cb = 2 * t + 1
        wait_chunk(cb, dstcb, srccb, semcb)
        cnt_b = scan_chunk(dstcb, srccb, dlcb, slcb)
        dump_fill(cnt_b, dlcb, slcb)
        fire_gathers((cnt_b + BG - 1) // BG, slcb, rowsb, semgb)

        @pl.when(t < NCHUNK // 2 - 1)
        def _():
            fire_chunk(cb + 2, dstcb, srccb, semcb)

        drain_and_rmw(cnt_a, dlca, slca, rowsa, semga)
        return cnt_b

    cnt_last = lax.fori_loop(0, NCHUNK // 2, pair_body, 0)
    drain_and_rmw(cnt_last, dlcb, slcb, rowsb, semgb)

    def merge_body(i, _):
        a0 = acc0[pl.ds(i * 16, 16)]
        a1 = acc1[pl.ds(i * 16, 16)]
        acc0[pl.ds(i * 16, 16)] = jnp.maximum(a0, a1)
        return 0

    lax.fori_loop(0, P, merge_body, 0)

    pltpu.sync_copy(acc0.at[pl.ds(0, P * DP)], m_hbm.at[wid])


@functools.partial(
    pl.kernel,
    out_type=jax.ShapeDtypeStruct((NW, P * DP), jnp.float32),
    mesh=plsc.VectorSubcoreMesh(core_axis_name="c", subcore_axis_name="s"),
    compiler_params=pltpu.CompilerParams(
        needs_layout_passes=False, use_tc_tiling_on_sc=False
    ),
    scratch_types=[
        pltpu.VMEM(((P + 1) * DP,), jnp.float32),
        pltpu.VMEM(((P + 1) * DP,), jnp.float32),
        pltpu.VMEM((CHUNK,), jnp.int32),
        pltpu.VMEM((CHUNK,), jnp.int32),
        pltpu.VMEM((CHUNK,), jnp.int32),
        pltpu.VMEM((CHUNK,), jnp.int32),
        pltpu.VMEM((COMP,), jnp.int32),
        pltpu.VMEM((COMP,), jnp.int32),
        pltpu.VMEM((COMP,), jnp.int32),
        pltpu.VMEM((COMP,), jnp.int32),
        pltpu.VMEM((WAVE * BG + 4, DP), jnp.float32),
        pltpu.VMEM((WAVE * BG + 4, DP), jnp.float32),
        pltpu.SemaphoreType.DMA,
        pltpu.SemaphoreType.DMA,
        pltpu.SemaphoreType.DMA,
        pltpu.SemaphoreType.DMA,
    ],
)
def _sc_segmax(
    y_hbm, dst_hbm, src_hbm, m_hbm,
    acc0, acc1, dstca, srcca, dstcb, srccb,
    dlca, slca, dlcb, slcb, rowsa, rowsb,
    semca, semcb, semga, semgb,
):
    _sc_body(
        y_hbm, dst_hbm, src_hbm, m_hbm,
        acc0, acc1, dstca, srcca, dstcb, srccb,
        dlca, slca, dlcb, slcb, rowsa, rowsb,
        semca, semcb, semga, semgb,
    )


@jax.jit
def kernel(x, edge_index, mask, W1, b1, W2, b2):
    del mask
    w1a = W1[:, :D]
    w1b = W1[:, D:]
    wy = jnp.zeros((DP, DP), jnp.float32).at[:D, :D].set(w1b.T)
    wz = jnp.zeros((DP, DP), jnp.float32).at[:D, :D].set((w1a - w1b).T)
    b1p = jnp.zeros((1, DP), jnp.float32).at[0, :D].set(b1)
    w2p = jnp.zeros((DP, DP), jnp.float32).at[:D, :D].set(W2.T)
    b2p = jnp.zeros((1, DP), jnp.float32).at[0, :D].set(b2)
    xp = jnp.pad(x, ((0, 0), (0, DP - D)))

    y16, z16 = _tc_pre(xp, wy, wz, b1p)
    src = edge_index[0].astype(jnp.int32)
    dst = edge_index[1].astype(jnp.int32)
    m = _sc_segmax(y16, dst, src)
    out16 = _tc_post(m.reshape(N, DP), z16, w2p, b2p)
    return out16[:, :D]


# speedup vs baseline: 2.0836x; 1.0230x over previous
import functools

import jax
import jax.numpy as jnp
from jax import lax
from jax.experimental import pallas as pl
from jax.experimental.pallas import tpu as pltpu
from jax.experimental.pallas import tpu_sc as plsc

N = 100000
E = 3200000
D = 10
DP = 16
NC = 2
NS = 16
NW = NC * NS
P = N // NW
CHUNK = 3200
BG = 128


def _tc_pre_body(x_ref, wy_ref, wz_ref, b1_ref, y_ref, z_ref):
    xb = x_ref[...]
    y_ref[...] = jnp.dot(xb, wy_ref[...], preferred_element_type=jnp.float32)
    z_ref[...] = (
        jnp.dot(xb, wz_ref[...], preferred_element_type=jnp.float32) + b1_ref[...]
    )


def _tc_pre(xp, wy, wz, b1p):
    br = 2000
    grid = N // br
    return pl.pallas_call(
        _tc_pre_body,
        grid=(grid,),
        in_specs=[
            pl.BlockSpec((br, D), lambda i: (i, 0)),
            pl.BlockSpec((D, DP), lambda i: (0, 0)),
            pl.BlockSpec((D, DP), lambda i: (0, 0)),
            pl.BlockSpec((1, DP), lambda i: (0, 0)),
        ],
        out_specs=[
            pl.BlockSpec((br, DP), lambda i: (i, 0)),
            pl.BlockSpec((br, DP), lambda i: (i, 0)),
        ],
        out_shape=[
            jax.ShapeDtypeStruct((N, DP), jnp.float32),
            jax.ShapeDtypeStruct((N, DP), jnp.float32),
        ],
    )(xp, wy, wz, b1p)


def _tc_post_body(m_ref, z_ref, w2_ref, b2_ref, o_ref):
    m = m_ref[...]
    agg = jnp.where(m == -jnp.inf, 0.0, z_ref[...] + m)
    o_ref[...] = (
        jnp.dot(agg, w2_ref[...], preferred_element_type=jnp.float32) + b2_ref[...]
    )


def _tc_post(m2, z16, w2p, b2p):
    br = 2000
    grid = N // br
    return pl.pallas_call(
        _tc_post_body,
        grid=(grid,),
        in_specs=[
            pl.BlockSpec((br, DP), lambda i: (i, 0)),
            pl.BlockSpec((br, DP), lambda i: (i, 0)),
            pl.BlockSpec((DP, D), lambda i: (0, 0)),
            pl.BlockSpec((1, D), lambda i: (0, 0)),
        ],
        out_specs=pl.BlockSpec((br, D), lambda i: (i, 0)),
        out_shape=jax.ShapeDtypeStruct((N, D), jnp.float32),
    )(m2, z16, w2p, b2p)


NCHUNK = E // CHUNK
WAVE = 1
COMP = CHUNK + 2 * BG
UNROLL = 8


def _sc_body(
    y_hbm, dst_hbm, src_hbm, m_hbm,
    acc0, acc1, dstca, srcca, dstcb, srccb,
    dlca, slca, dlcb, slcb, rowsa, rowsb,
    semca, semcb, semga, semgb,
):
    cid = lax.axis_index("c")
    sid = lax.axis_index("s")
    wid = sid * NC + cid
    base = wid * P
    iota = lax.iota(jnp.int32, 16)
    neg_inf = jnp.full((16,), -jnp.inf, jnp.float32)
    dump_row = jnp.full((16,), P, jnp.int32)
    ones = jnp.full((16,), 1, jnp.int32)
    zeros = jnp.full((16,), 0, jnp.int32)
    pu = jnp.uint32(P)

    def init_body(i, _):
        plsc.store_scatter(acc0, [i * 16 + iota], neg_inf)
        plsc.store_scatter(acc1, [i * 16 + iota], neg_inf)
        return 0

    lax.fori_loop(0, P + 1, init_body, 0)

    def fire_chunk(c, dref, sref, semc):
        off = c * CHUNK
        pltpu.async_copy(dst_hbm.at[pl.ds(off, CHUNK)], dref, semc)
        pltpu.async_copy(src_hbm.at[pl.ds(off, CHUNK)], sref, semc)

    def wait_chunk(c, dref, sref, semc):
        off = c * CHUNK
        pltpu.make_async_copy(dst_hbm.at[pl.ds(off, CHUNK)], dref, semc).wait()
        pltpu.make_async_copy(src_hbm.at[pl.ds(off, CHUNK)], sref, semc).wait()

    lane15 = jnp.full((16,), 15, jnp.int32)

    def bcast_last(vec):
        return lax.gather(
            vec,
            lane15[:, None],
            dimension_numbers=lax.GatherDimensionNumbers(
                offset_dims=(), collapsed_slice_dims=(0,), start_index_map=(0,)
            ),
            slice_sizes=(1,),
            mode=lax.GatherScatterMode.PROMISE_IN_BOUNDS,
        )

    def scan_chunk(dref, sref, dlc, slc):
        def scan_body(g4, cntv):
            g = g4 * (16 * UNROLL)
            parts = []
            for k in range(UNROLL):
                d16 = dref[pl.ds(g + k * 16, 16)]
                s16 = sref[pl.ds(g + k * 16, 16)]
                dl = d16 - base
                msk = dl.astype(jnp.uint32) < pu
                mi = jnp.where(msk, ones, zeros)
                pos = plsc.cumsum(mi)
                parts.append((dl, s16, msk, pos))
            for dl, s16, msk, pos in parts:
                offs = pos + cntv
                plsc.store_scatter(dlc, [offs], dl, mask=msk)
                plsc.store_scatter(slc, [offs], s16, mask=msk)
                cntv = cntv + bcast_last(pos)
            return cntv

        cntv = lax.fori_loop(
            0, CHUNK // (16 * UNROLL), scan_body,
            jnp.full((16,), -1, jnp.int32),
        )
        return jnp.max(cntv) + 1

    def dump_fill(cnt, dlc, slc):
        def dump_body(j, _):
            idx16 = cnt + j * 16 + iota
            plsc.store_scatter(dlc, [idx16], dump_row)
            plsc.store_scatter(slc, [idx16], (idx16 * 1237 + wid * 61) & 0xFFFF)
            return 0

        lax.fori_loop(0, BG // 16, dump_body, 0)

    def fire_gathers(nblk, slc, rows, semg):
        nb = jnp.minimum(nblk, WAVE)

        def fire_blk(k, _):
            pltpu.async_copy(
                y_hbm.at[slc.at[pl.ds(k * BG, BG)]],
                rows.at[pl.ds(k * BG, BG)],
                semg,
            )
            return 0

        lax.fori_loop(0, nb, fire_blk, 0)

    def rmw_edges(b0, nb, cnt, dlc, rows):
        npairs = (jnp.clip(cnt - b0 * BG, 0, nb * BG) + 1) // 2
        nquad = (npairs + 3) // 4

        def edge_body(j, _):
            for q in range(4):
                e = b0 * BG + 8 * j + 2 * q
                r = 8 * j + 2 * q
                dl0 = plsc.load_gather(dlc, [jnp.full((16,), e, jnp.int32)])
                dl1 = plsc.load_gather(dlc, [jnp.full((16,), e + 1, jnp.int32)])
                msg0 = rows[r]
                msg1 = rows[r + 1]
                f0 = dl0 * 16 + iota
                f1 = dl1 * 16 + iota
                cur0 = plsc.load_gather(acc0, [f0])
                cur1 = plsc.load_gather(acc1, [f1])
                plsc.store_scatter(acc0, [f0], jnp.maximum(cur0, msg0))
                plsc.store_scatter(acc1, [f1], jnp.maximum(cur1, msg1))
            return 0

        lax.fori_loop(0, nquad, edge_body, 0)

    def drain_and_rmw(cnt, dlc, slc, rows, semg):
        nblk = (cnt + BG - 1) // BG
        nb0 = jnp.minimum(nblk, WAVE)

        def drain_blk(k, _):
            pltpu.make_async_copy(
                y_hbm.at[slc.at[pl.ds(k * BG, BG)]],
                rows.at[pl.ds(k * BG, BG)],
                semg,
            ).wait()
            return 0

        lax.fori_loop(0, nb0, drain_blk, 0)
        rmw_edges(0, nb0, cnt, dlc, rows)

        def over_body(b, _):
            pltpu.async_copy(
                y_hbm.at[slc.at[pl.ds(b * BG, BG)]],
                rows.at[pl.ds(0, BG)],
                semg,
            ).wait()
            npairs = (jnp.clip(cnt - b * BG, 0, BG) + 1) // 2

            def edge_body(j, _):
                e = b * BG + 2 * j
                dl0 = plsc.load_gather(dlc, [jnp.full((16,), e, jnp.int32)])
                dl1 = plsc.load_gather(dlc, [jnp.full((16,), e + 1, jnp.int32)])
                msg0 = rows[2 * j]
                msg1 = rows[2 * j + 1]
                f0 = dl0 * 16 + iota
                f1 = dl1 * 16 + iota
                cur0 = plsc.load_gather(acc0, [f0])
                cur1 = plsc.load_gather(acc1, [f1])
                plsc.store_scatter(acc0, [f0], jnp.maximum(cur0, msg0))
                plsc.store_scatter(acc1, [f1], jnp.maximum(cur1, msg1))
                return 0

            lax.fori_loop(0, npairs, edge_body, 0)
            return 0

        lax.fori_loop(WAVE, nblk, over_body, 0)

    fire_chunk(0, dstca, srcca, semca)
    fire_chunk(1, dstcb, srccb, semcb)

    def pair_body(t, cnt_b):
        ca = 2 * t
        wait_chunk(ca, dstca, srcca, semca)
        cnt_a = scan_chunk(dstca, srcca, dlca, slca)
        dump_fill(cnt_a, dlca, slca)
        fire_gathers((cnt_a + BG - 1) // BG, slca, rowsa, semga)

        @pl.when(t < NCHUNK // 2 - 1)
        def _():
            fire_chunk(ca + 2, dstca, srcca, semca)

        drain_and_rmw(cnt_b, dlcb, slcb, rowsb, semgb)

        cb = 2 * t + 1
        wait_chunk(cb, dstcb, srccb, semcb)
        cnt_b = scan_chunk(dstcb, srccb, dlcb, slcb)
        dump_fill(cnt_b, dlcb, slcb)
        fire_gathers((cnt_b + BG - 1) // BG, slcb, rowsb, semgb)

        @pl.when(t < NCHUNK // 2 - 1)
        def _():
            fire_chunk(cb + 2, dstcb, srccb, semcb)

        drain_and_rmw(cnt_a, dlca, slca, rowsa, semga)
        return cnt_b

    cnt_last = lax.fori_loop(0, NCHUNK // 2, pair_body, 0)
    drain_and_rmw(cnt_last, dlcb, slcb, rowsb, semgb)

    def merge_body(i, _):
        a0 = acc0[pl.ds(i * 16, 16)]
        a1 = acc1[pl.ds(i * 16, 16)]
        acc0[pl.ds(i * 16, 16)] = jnp.maximum(a0, a1)
        return 0

    lax.fori_loop(0, P, merge_body, 0)

    pltpu.sync_copy(acc0.at[pl.ds(0, P * DP)], m_hbm.at[wid])


@functools.partial(
    pl.kernel,
    out_type=jax.ShapeDtypeStruct((NW, P * DP), jnp.float32),
    mesh=plsc.VectorSubcoreMesh(core_axis_name="c", subcore_axis_name="s"),
    compiler_params=pltpu.CompilerParams(
        needs_layout_passes=False, use_tc_tiling_on_sc=False
    ),
    scratch_types=[
        pltpu.VMEM(((P + 1) * DP,), jnp.float32),
        pltpu.VMEM(((P + 1) * DP,), jnp.float32),
        pltpu.VMEM((CHUNK,), jnp.int32),
        pltpu.VMEM((CHUNK,), jnp.int32),
        pltpu.VMEM((CHUNK,), jnp.int32),
        pltpu.VMEM((CHUNK,), jnp.int32),
        pltpu.VMEM((COMP,), jnp.int32),
        pltpu.VMEM((COMP,), jnp.int32),
        pltpu.VMEM((COMP,), jnp.int32),
        pltpu.VMEM((COMP,), jnp.int32),
        pltpu.VMEM((WAVE * BG + 8, DP), jnp.float32),
        pltpu.VMEM((WAVE * BG + 8, DP), jnp.float32),
        pltpu.SemaphoreType.DMA,
        pltpu.SemaphoreType.DMA,
        pltpu.SemaphoreType.DMA,
        pltpu.SemaphoreType.DMA,
    ],
)
def _sc_segmax(
    y_hbm, dst_hbm, src_hbm, m_hbm,
    acc0, acc1, dstca, srcca, dstcb, srccb,
    dlca, slca, dlcb, slcb, rowsa, rowsb,
    semca, semcb, semga, semgb,
):
    _sc_body(
        y_hbm, dst_hbm, src_hbm, m_hbm,
        acc0, acc1, dstca, srcca, dstcb, srccb,
        dlca, slca, dlcb, slcb, rowsa, rowsb,
        semca, semcb, semga, semgb,
    )


@jax.jit
def kernel(x, edge_index, mask, W1, b1, W2, b2):
    del mask
    w1a = W1[:, :D]
    w1b = W1[:, D:]
    wy = jnp.zeros((D, DP), jnp.float32).at[:, :D].set(w1b.T)
    wz = jnp.zeros((D, DP), jnp.float32).at[:, :D].set((w1a - w1b).T)
    b1p = jnp.zeros((1, DP), jnp.float32).at[0, :D].set(b1)
    w2p = jnp.zeros((DP, D), jnp.float32).at[:D, :].set(W2.T)
    b2p = b2.reshape(1, D)

    y16, z16 = _tc_pre(x, wy, wz, b1p)
    src = edge_index[0].astype(jnp.int32)
    dst = edge_index[1].astype(jnp.int32)
    m = _sc_segmax(y16, dst, src)
    return _tc_post(m.reshape(N, DP), z16, w2p, b2p)
